# baseline (speedup 1.0000x reference)
import functools

import jax
import jax.numpy as jnp
from jax.experimental import pallas as pl
from jax.experimental.pallas import tpu as pltpu

_L = 3
_C0 = 3
_HID = 64
_HW = 128
_BG = 2


def _leaky(v):
    return jnp.maximum(v, 0.1 * v)


def _pool_lanes(z, hid):
    m01 = jnp.maximum(z[:, 0:hid], z[:, hid:2 * hid])
    m23 = jnp.maximum(z[:, 2 * hid:3 * hid], z[:, 3 * hid:4 * hid])
    return jnp.maximum(m01, m23)


_PAR = [(1, 0), (0, 1), (1, 1), (0, 2)]


def _net_kernel(p4_ref, w4_ref, s4_ref, b4_ref,
                wt4_ref, st4_ref, bt4_ref, out_ref,
                pp_ref, yr_ref, lhs_ref, *, bg, hid):
    s4 = s4_ref[0]
    b4 = b4_ref[0]

    for g in range(bg):
        acc = jax.lax.dot_general(
            p4_ref[0, 0, :, g * 4096:(g + 1) * 4096], w4_ref[0],
            (((0,), (0,)), ((), ())),
            preferred_element_type=jnp.float32)
        y = _leaky(_pool_lanes(acc * s4 + b4, hid))
        for ry in range(2):
            for rx in range(2):
                seg = y[(ry * 2 + rx) * 1024:(ry * 2 + rx + 1) * 1024]
                pp_ref[g, ry, rx, 1:33, 1:33, :] = seg.reshape(32, 32, hid)

    sh = 32
    for k in range(6):
        p = sh * sh
        for g in range(bg):
            for ry in range(2):
                for rx in range(2):
                    ze = jnp.zeros((sh + 2, hid), jnp.float32)
                    pp_ref[g, ry, rx, 0, 0:sh + 2, :] = ze
                    pp_ref[g, ry, rx, sh + 1, 0:sh + 2, :] = ze
                    pp_ref[g, ry, rx, 1:sh + 1, 0, :] = ze[0:sh]
                    pp_ref[g, ry, rx, 1:sh + 1, sh + 1, :] = ze[0:sh]
        for g in range(bg):
            for iy in range(4):
                py, oy = _PAR[iy]
                for jx in range(2):
                    ix0, ix1 = 2 * jx, 2 * jx + 1
                    px0, ox0 = _PAR[ix0]
                    px1, ox1 = _PAR[ix1]
                    sl0 = pp_ref[g, py, px0, oy:oy + sh, ox0:ox0 + sh, :]
                    sl1 = pp_ref[g, py, px1, oy:oy + sh, ox1:ox1 + sh, :]
                    pair = jnp.concatenate(
                        [sl0.reshape(p, hid), sl1.reshape(p, hid)],
                        axis=1).astype(jnp.bfloat16)
                    col = (iy * 4 + 2 * jx) * hid
                    lhs_ref[g * p:(g + 1) * p, col:col + 2 * hid] = pair
        acc = jnp.dot(lhs_ref[0:bg * p, :], wt4_ref[0, k],
                      preferred_element_type=jnp.float32)
        y = _leaky(_pool_lanes(acc * st4_ref[0, k] + bt4_ref[0, k], hid))
        if k < 5:
            nh = sh // 2
            for g in range(bg):
                yr_ref[g, 0:sh, 0:sh, :] = (
                    y[g * p:(g + 1) * p].reshape(sh, sh, hid))
            for g in range(bg):
                for ry in range(2):
                    for rx in range(2):
                        pl_ = yr_ref[g, pl.ds(ry, nh, 2), pl.ds(rx, nh, 2), :]
                        pp_ref[g, ry, rx, 1:nh + 1, 1:nh + 1, :] = pl_
            sh = nh
        else:
            out_ref[0] = y


def _expand_w4(w9, hid):
    parts = [jnp.pad(w9, ((py, 1 - py), (px, 1 - px), (0, 0), (0, 0)))
             for py in (0, 1) for px in (0, 1)]
    w4 = jnp.stack(parts, axis=-2)
    cin = w9.shape[2]
    return w4.reshape(16 * cin, 4 * hid)


def kernel(x, w0_0, s0_0, b0_0, wt_0, st_0, bt_0,
           w0_1, s0_1, b0_1, wt_1, st_1, bt_1,
           w0_2, s0_2, b0_2, wt_2, st_2, bt_2, wc, bc):
    L, C0, HID, HW, BG = _L, _C0, _HID, _HW, _BG
    B = x.shape[0]
    NS = B // BG

    x5 = x.reshape(B, L, C0, HW, HW)
    xp = jnp.pad(x5, ((0, 0), (0, 0), (0, 0), (1, 1), (1, 1)))
    slabs = [xp[:, :, :, 2 * ry + iy:2 * ry + iy + HW:4,
                2 * rx + ix:2 * rx + ix + HW:4]
             for iy in range(4) for ix in range(4)
             for ry in range(2) for rx in range(2)]
    p4 = jnp.stack(slabs, axis=0)
    p4 = p4.reshape(16, 2, 2, B, L, C0, 32, 32)
    p4 = p4.transpose(3, 4, 0, 5, 1, 2, 6, 7)
    p4 = p4.reshape(NS, BG, L, 48, 4096).transpose(0, 2, 3, 1, 4)
    p4 = p4.reshape(NS, L, 48, BG * 4096).astype(jnp.bfloat16)

    w4 = jnp.stack([_expand_w4(w.reshape(3, 3, C0, HID), HID)
                    for w in (w0_0, w0_1, w0_2)]).astype(jnp.bfloat16)
    s4 = jnp.stack([jnp.tile(s, (1, 4)) for s in (s0_0, s0_1, s0_2)])
    b4 = jnp.stack([jnp.tile(b, (1, 4)) for b in (b0_0, b0_1, b0_2)])
    wt4 = jnp.stack(
        [jnp.stack([_expand_w4(wt[k].reshape(3, 3, HID, HID), HID)
                    for k in range(6)]) for wt in (wt_0, wt_1, wt_2)]
    ).astype(jnp.bfloat16)
    st4 = jnp.stack([jnp.tile(s, (1, 1, 4)) for s in (st_0, st_1, st_2)])
    bt4 = jnp.stack([jnp.tile(b, (1, 1, 4)) for b in (bt_0, bt_1, bt_2)])

    feats = pl.pallas_call(
        functools.partial(_net_kernel, bg=BG, hid=HID),
        out_shape=jax.ShapeDtypeStruct((L * NS, BG, HID), jnp.float32),
        grid=(L * NS,),
        in_specs=[
            pl.BlockSpec((1, 1, 48, BG * 4096),
                         lambda i: (i % NS, i // NS, 0, 0)),
            pl.BlockSpec((1, 48, 4 * HID), lambda i: (i // NS, 0, 0)),
            pl.BlockSpec((1, 1, 4 * HID), lambda i: (i // NS, 0, 0)),
            pl.BlockSpec((1, 1, 4 * HID), lambda i: (i // NS, 0, 0)),
            pl.BlockSpec((1, 6, 16 * HID, 4 * HID),
                         lambda i: (i // NS, 0, 0, 0)),
            pl.BlockSpec((1, 6, 1, 4 * HID), lambda i: (i // NS, 0, 0, 0)),
            pl.BlockSpec((1, 6, 1, 4 * HID), lambda i: (i // NS, 0, 0, 0)),
        ],
        out_specs=pl.BlockSpec((1, BG, HID), lambda i: (i, 0, 0)),
        scratch_shapes=[
            pltpu.VMEM((BG, 2, 2, 34, 34, HID), jnp.float32),
            pltpu.VMEM((BG, 32, 32, HID), jnp.float32),
            pltpu.VMEM((BG * 1024, 16 * HID), jnp.bfloat16),
        ],
        compiler_params=pltpu.CompilerParams(
            dimension_semantics=("parallel",),
            vmem_limit_bytes=64 * 1024 * 1024),
    )(p4, w4, s4, b4, wt4, st4, bt4)

    f = feats.reshape(L, B, HID).transpose(1, 0, 2).reshape(B, L * HID)
    out = jnp.dot(f, wc, precision=jax.lax.Precision.HIGHEST) + bc
    return out.reshape(B, 1, 1, 1)


# speedup vs baseline: 3.2887x; 3.2887x over previous
import functools

import jax
import jax.numpy as jnp
import numpy as np
from jax.experimental import pallas as pl
from jax.experimental.pallas import tpu as pltpu

_L = 3
_C0 = 3
_HID = 64
_HW = 128
_BG = 2
_KH = 768

_PAR = [(1, 0), (0, 1), (1, 1), (0, 2)]
_SH = [32, 16, 8, 4, 2, 1]
_OFF = [0, 34, 52, 62, 68, 72, 75]


def _leaky(v):
    return jnp.maximum(v, 0.1 * v)


def _band_sel():
    e = np.zeros((2, 3, 72, 64), np.float32)
    for h in range(2):
        base = 56 * h
        for dx in range(3):
            for w in range(64):
                wp = w + 64 * h + dx - 1
                r = wp - base
                if 0 <= r < 72 and 0 <= wp < _HW:
                    wcol = (w & 1) * 32 + ((w >> 1) & 1) * 16 + (w >> 2)
                    e[h, dx, r, wcol] = 1.0
    return e


def _net_kernel(x_ref, bd0_ref, bd1_ref, s1_ref, b1_ref,
                wt4_ref, st4_ref, bt4_ref, out_ref,
                xpad_ref, lhs1_ref, pp_ref, yr_ref, lhs_ref, *, bg, hid):
    xpad_ref[:, :, 0, :] = jnp.zeros((bg, _C0, _HW), jnp.float32)
    xpad_ref[:, :, _HW + 1, :] = jnp.zeros((bg, _C0, _HW), jnp.float32)
    for g in range(bg):
        for c in range(_C0):
            xpad_ref[g, c, 1:_HW + 1, :] = x_ref[g, c]

    zpad = jnp.zeros((bg * 128, _KH - 648), jnp.bfloat16)
    lhs1_ref[:, 648:_KH] = zpad
    lhs1_ref[:, _KH + 648:2 * _KH] = zpad
    for g in range(bg):
        for hp in range(2):
            for ry in range(2):
                row = g * 128 + hp * 64 + ry * 32
                for c in range(_C0):
                    for dy in range(3):
                        src = xpad_ref[g, c, pl.ds(2 * ry + hp + dy, 32, 4), :]
                        col = (c * 3 + dy) * 72
                        lhs1_ref[row:row + 32, col:col + 72] = (
                            src[:, 0:72].astype(jnp.bfloat16))
                        lhs1_ref[row:row + 32, _KH + col:_KH + col + 72] = (
                            src[:, 56:128].astype(jnp.bfloat16))

    pp_ref[:, :, :, 0, :, :] = jnp.zeros((bg, 2, 2, _OFF[6], hid), jnp.bfloat16)
    for k in range(6):
        sh, ok = _SH[k], _OFF[k]
        z2 = jnp.zeros((bg, 2, 2, sh + 2, hid), jnp.bfloat16)
        pp_ref[:, :, :, sh + 1, ok:ok + sh + 2, :] = z2
        pp_ref[:, :, :, 1:sh + 1, ok, :] = z2[:, :, :, 0:sh]
        pp_ref[:, :, :, 1:sh + 1, ok + sh + 1, :] = z2[:, :, :, 0:sh]

    s1 = s1_ref[0]
    b1 = b1_ref[0]
    for half, bd in ((0, bd0_ref), (1, bd1_ref)):
        z = jnp.dot(lhs1_ref[:, half * _KH:(half + 1) * _KH], bd[0, 0],
                    preferred_element_type=jnp.float32)
        z = z * s1 + b1
        m = jnp.maximum(z[:, 0:2048], z[:, 2048:4096])
        for g in range(bg):
            mg = jnp.maximum(m[g * 128:g * 128 + 64],
                             m[g * 128 + 64:(g + 1) * 128])
            y = _leaky(mg)
            for ry in range(2):
                for rx in range(2):
                    seg = y[ry * 32:(ry + 1) * 32,
                            rx * 1024:(rx + 1) * 1024].reshape(32, 16, hid)
                    pp_ref[g, ry, rx, 1:33,
                           1 + 16 * half:17 + 16 * half, :] = (
                        seg.astype(jnp.bfloat16))

    for k in range(6):
        sh, ok = _SH[k], _OFF[k]
        p = sh * sh
        for g in range(bg):
            for iy in range(4):
                py, oy = _PAR[iy]
                for jx in range(2):
                    px0, ox0 = _PAR[2 * jx]
                    px1, ox1 = _PAR[2 * jx + 1]
                    sl0 = pp_ref[g, py, px0, oy:oy + sh,
                                 ok + ox0:ok + ox0 + sh, :]
                    sl1 = pp_ref[g, py, px1, oy:oy + sh,
                                 ok + ox1:ok + ox1 + sh, :]
                    pair = jnp.concatenate(
                        [sl0.reshape(p, hid), sl1.reshape(p, hid)], axis=1)
                    col = (iy * 4 + 2 * jx) * hid
                    lhs_ref[g * p:(g + 1) * p, col:col + 2 * hid] = pair
        acc = jnp.dot(lhs_ref[0:bg * p, :], wt4_ref[0, k],
                      preferred_element_type=jnp.float32)
        zt = acc * st4_ref[0, k] + bt4_ref[0, k]
        m = jnp.maximum(jnp.maximum(zt[:, 0:hid], zt[:, hid:2 * hid]),
                        jnp.maximum(zt[:, 2 * hid:3 * hid],
                                    zt[:, 3 * hid:4 * hid]))
        y = _leaky(m)
        if k < 5:
            nh = sh // 2
            for g in range(bg):
                yr_ref[g, 0:sh, 0:sh, :] = (
                    y[g * p:(g + 1) * p].reshape(sh, sh, hid))
            for g in range(bg):
                for ry in range(2):
                    for rx in range(2):
                        pln = yr_ref[g, pl.ds(ry, nh, 2), pl.ds(rx, nh, 2), :]
                        pp_ref[g, ry, rx, 1:nh + 1,
                               _OFF[k + 1] + 1:_OFF[k + 1] + 1 + nh, :] = (
                            pln.astype(jnp.bfloat16))
        else:
            out_ref[0] = y


def _expand_w4(w9, hid):
    parts = [jnp.pad(w9, ((py, 1 - py), (px, 1 - px), (0, 0), (0, 0)))
             for py in (0, 1) for px in (0, 1)]
    w4 = jnp.stack(parts, axis=-2)
    cin = w9.shape[2]
    return w4.reshape(16 * cin, 4 * hid)


def kernel(x, w0_0, s0_0, b0_0, wt_0, st_0, bt_0,
           w0_1, s0_1, b0_1, wt_1, st_1, bt_1,
           w0_2, s0_2, b0_2, wt_2, st_2, bt_2, wc, bc):
    L, C0, HID, HW, BG = _L, _C0, _HID, _HW, _BG
    B = x.shape[0]
    NS = B // BG

    w9s = jnp.stack([w.reshape(3, 3, C0, HID)
                     for w in (w0_0, w0_1, w0_2)])
    esel = jnp.asarray(_band_sel())
    band = jnp.einsum('hxrw,lyxco->lhcyrwo', esel, w9s)
    band = band.reshape(L, 2, 9 * 72, 64 * HID)
    band = jnp.pad(band, ((0, 0), (0, 0), (0, _KH - 648), (0, 0)))
    band = band.astype(jnp.bfloat16)
    s1t = jnp.stack([jnp.tile(s, (1, 64)) for s in (s0_0, s0_1, s0_2)])
    b1t = jnp.stack([jnp.tile(b, (1, 64)) for b in (b0_0, b0_1, b0_2)])

    wt4 = jnp.stack(
        [jnp.stack([_expand_w4(wt[k].reshape(3, 3, HID, HID), HID)
                    for k in range(6)])
         for wt in (wt_0, wt_1, wt_2)]
    ).astype(jnp.bfloat16)
    st4 = jnp.stack([jnp.tile(s, (1, 1, 4)) for s in (st_0, st_1, st_2)])
    bt4 = jnp.stack([jnp.tile(b, (1, 1, 4)) for b in (bt_0, bt_1, bt_2)])

    feats = pl.pallas_call(
        functools.partial(_net_kernel, bg=BG, hid=HID),
        out_shape=jax.ShapeDtypeStruct((L * NS, BG, HID), jnp.float32),
        grid=(L * NS,),
        in_specs=[
            pl.BlockSpec((BG, C0, HW, HW), lambda i: (i % NS, i // NS, 0, 0)),
            pl.BlockSpec((1, 1, _KH, 64 * HID), lambda i: (i // NS, 0, 0, 0)),
            pl.BlockSpec((1, 1, _KH, 64 * HID), lambda i: (i // NS, 1, 0, 0)),
            pl.BlockSpec((1, 1, 64 * HID), lambda i: (i // NS, 0, 0)),
            pl.BlockSpec((1, 1, 64 * HID), lambda i: (i // NS, 0, 0)),
            pl.BlockSpec((1, 6, 16 * HID, 4 * HID),
                         lambda i: (i // NS, 0, 0, 0)),
            pl.BlockSpec((1, 6, 1, 4 * HID), lambda i: (i // NS, 0, 0, 0)),
            pl.BlockSpec((1, 6, 1, 4 * HID), lambda i: (i // NS, 0, 0, 0)),
        ],
        out_specs=pl.BlockSpec((1, BG, HID), lambda i: (i, 0, 0)),
        scratch_shapes=[
            pltpu.VMEM((BG, C0, HW + 2, HW), jnp.float32),
            pltpu.VMEM((BG * 128, 2 * _KH), jnp.bfloat16),
            pltpu.VMEM((BG, 2, 2, 34, _OFF[6], HID), jnp.bfloat16),
            pltpu.VMEM((BG, 32, 32, HID), jnp.float32),
            pltpu.VMEM((BG * 1024, 16 * HID), jnp.bfloat16),
        ],
        compiler_params=pltpu.CompilerParams(
            dimension_semantics=("parallel",),
            vmem_limit_bytes=64 * 1024 * 1024),
    )(x, band, band, s1t, b1t, wt4, st4, bt4)

    f = feats.reshape(L, B, HID).transpose(1, 0, 2).reshape(B, L * HID)
    out = jnp.dot(f, wc, precision=jax.lax.Precision.HIGHEST) + bc
    return out.reshape(B, 1, 1, 1)


# speedup vs baseline: 4.1771x; 1.2701x over previous
import functools

import jax
import jax.numpy as jnp
import numpy as np
from jax.experimental import pallas as pl
from jax.experimental.pallas import tpu as pltpu

_L = 3
_C0 = 3
_HID = 64
_HW = 128
_BG = 2
_KH = 768

_PAR = [(1, 0), (0, 1), (1, 1), (0, 2)]
_SH = [32, 16, 8, 4, 2, 1]
_OFF = [0, 34, 52, 62, 68, 72, 75]


def _leaky(v):
    return jnp.maximum(v, 0.1 * v)


def _band_sel():
    e = np.zeros((2, 3, 72, 64), np.float32)
    for h in range(2):
        base = 56 * h
        for dx in range(3):
            for w in range(64):
                wp = w + 64 * h + dx - 1
                r = wp - base
                if 0 <= r < 72 and 0 <= wp < _HW:
                    wcol = (w & 1) * 32 + (w >> 2) * 2 + ((w >> 1) & 1)
                    e[h, dx, r, wcol] = 1.0
    return e


def _net_kernel(x_ref, bd0_ref, bd1_ref, s1_ref, b1_ref,
                wt4_ref, st4_ref, bt4_ref, out_ref,
                xpad_ref, lhs1_ref, pp_ref, yr_ref, lhs_ref, *, bg, hid):
    xpad_ref[:, :, 0, :] = jnp.zeros((bg, _C0, _HW), jnp.float32)
    xpad_ref[:, :, _HW + 1, :] = jnp.zeros((bg, _C0, _HW), jnp.float32)
    for g in range(bg):
        for c in range(_C0):
            xpad_ref[g, c, 1:_HW + 1, :] = x_ref[g, c]

    zpad = jnp.zeros((bg * 128, _KH - 648), jnp.bfloat16)
    lhs1_ref[:, 648:_KH] = zpad
    lhs1_ref[:, _KH + 648:2 * _KH] = zpad
    for g in range(bg):
        for hp in range(2):
            for ry in range(2):
                row = g * 128 + hp * 64 + ry * 32
                for c in range(_C0):
                    for dy in range(3):
                        src = xpad_ref[g, c, pl.ds(2 * ry + hp + dy, 32, 4), :]
                        col = (c * 3 + dy) * 72
                        lhs1_ref[row:row + 32, col:col + 72] = (
                            src[:, 0:72].astype(jnp.bfloat16))
                        lhs1_ref[row:row + 32, _KH + col:_KH + col + 72] = (
                            src[:, 56:128].astype(jnp.bfloat16))

    pp_ref[:, :, 0, :, :] = jnp.zeros((bg, 2, _OFF[6], 2 * hid), jnp.float32)
    for k in range(6):
        sh, ok = _SH[k], _OFF[k]
        z2 = jnp.zeros((bg, 2, sh + 2, 2 * hid), jnp.float32)
        pp_ref[:, :, sh + 1, ok:ok + sh + 2, :] = z2
        pp_ref[:, :, 1:sh + 1, ok, :] = z2[:, :, 0:sh]
        pp_ref[:, :, 1:sh + 1, ok + sh + 1, :] = z2[:, :, 0:sh]

    s1 = s1_ref[0]
    b1 = b1_ref[0]
    for half, bd in ((0, bd0_ref), (1, bd1_ref)):
        z = jnp.dot(lhs1_ref[:, half * _KH:(half + 1) * _KH], bd[0, 0],
                    preferred_element_type=jnp.float32)
        z = z * s1 + b1
        m = jnp.maximum(z[:, 0:2048], z[:, 2048:4096])
        for g in range(bg):
            mg = jnp.maximum(m[g * 128:g * 128 + 64],
                             m[g * 128 + 64:(g + 1) * 128])
            y = _leaky(mg)
            for ry in range(2):
                pp_ref[g, ry, 1:33, 1 + 16 * half:17 + 16 * half, :] = (
                    y[ry * 32:(ry + 1) * 32].reshape(32, 16, 2 * hid))

    for k in range(6):
        sh, ok = _SH[k], _OFF[k]
        p = sh * sh
        for g in range(bg):
            for iy in range(4):
                py, oy = _PAR[iy]
                rows = pl.ds(g * p, p)
                mid = pp_ref[g, py, oy:oy + sh, ok + 1:ok + 1 + sh, :]
                lhs_ref[rows, iy * 256 + 64:iy * 256 + 192] = (
                    mid.reshape(p, 2 * hid).astype(jnp.bfloat16))
                s0 = pp_ref[g, py, oy:oy + sh, ok:ok + sh, hid:2 * hid]
                lhs_ref[rows, iy * 256:iy * 256 + 64] = (
                    s0.reshape(p, hid).astype(jnp.bfloat16))
                s3 = pp_ref[g, py, oy:oy + sh, ok + 2:ok + 2 + sh, 0:hid]
                lhs_ref[rows, iy * 256 + 192:iy * 256 + 256] = (
                    s3.reshape(p, hid).astype(jnp.bfloat16))
        acc = jnp.dot(lhs_ref[0:bg * p, :], wt4_ref[0, k],
                      preferred_element_type=jnp.float32)
        zt = acc * st4_ref[0, k] + bt4_ref[0, k]
        m = jnp.maximum(jnp.maximum(zt[:, 0:hid], zt[:, hid:2 * hid]),
                        jnp.maximum(zt[:, 2 * hid:3 * hid],
                                    zt[:, 3 * hid:4 * hid]))
        y = _leaky(m)
        if k < 5:
            nh = sh // 2
            for g in range(bg):
                yr_ref[g, 0:sh, 0:sh, :] = (
                    y[g * p:(g + 1) * p].reshape(sh, sh, hid))
            for g in range(bg):
                for ry in range(2):
                    for rx in range(2):
                        pln = yr_ref[g, pl.ds(ry, nh, 2), pl.ds(rx, nh, 2), :]
                        pp_ref[g, ry, 1:nh + 1,
                               _OFF[k + 1] + 1:_OFF[k + 1] + 1 + nh,
                               rx * hid:(rx + 1) * hid] = pln
        else:
            out_ref[0] = y


def _expand_w4(w9, hid):
    parts = [jnp.pad(w9, ((py, 1 - py), (px, 1 - px), (0, 0), (0, 0)))
             for py in (0, 1) for px in (0, 1)]
    w4 = jnp.stack(parts, axis=-2)
    cin = w9.shape[2]
    return w4.reshape(16 * cin, 4 * hid)


def kernel(x, w0_0, s0_0, b0_0, wt_0, st_0, bt_0,
           w0_1, s0_1, b0_1, wt_1, st_1, bt_1,
           w0_2, s0_2, b0_2, wt_2, st_2, bt_2, wc, bc):
    L, C0, HID, HW, BG = _L, _C0, _HID, _HW, _BG
    B = x.shape[0]
    NS = B // BG

    w9s = jnp.stack([w.reshape(3, 3, C0, HID)
                     for w in (w0_0, w0_1, w0_2)])
    esel = jnp.asarray(_band_sel())
    band = jnp.einsum('hxrw,lyxco->lhcyrwo', esel, w9s)
    band = band.reshape(L, 2, 9 * 72, 64 * HID)
    band = jnp.pad(band, ((0, 0), (0, 0), (0, _KH - 648), (0, 0)))
    band = band.astype(jnp.bfloat16)
    s1t = jnp.stack([jnp.tile(s, (1, 64)) for s in (s0_0, s0_1, s0_2)])
    b1t = jnp.stack([jnp.tile(b, (1, 64)) for b in (b0_0, b0_1, b0_2)])

    wt4 = jnp.stack(
        [jnp.stack([_expand_w4(wt[k].reshape(3, 3, HID, HID), HID)
                    for k in range(6)])
         for wt in (wt_0, wt_1, wt_2)]
    ).astype(jnp.bfloat16)
    st4 = jnp.stack([jnp.tile(s, (1, 1, 4)) for s in (st_0, st_1, st_2)])
    bt4 = jnp.stack([jnp.tile(b, (1, 1, 4)) for b in (bt_0, bt_1, bt_2)])

    feats = pl.pallas_call(
        functools.partial(_net_kernel, bg=BG, hid=HID),
        out_shape=jax.ShapeDtypeStruct((L * NS, BG, HID), jnp.float32),
        grid=(L * NS,),
        in_specs=[
            pl.BlockSpec((BG, C0, HW, HW), lambda i: (i % NS, i // NS, 0, 0)),
            pl.BlockSpec((1, 1, _KH, 64 * HID), lambda i: (i // NS, 0, 0, 0)),
            pl.BlockSpec((1, 1, _KH, 64 * HID), lambda i: (i // NS, 1, 0, 0)),
            pl.BlockSpec((1, 1, 64 * HID), lambda i: (i // NS, 0, 0)),
            pl.BlockSpec((1, 1, 64 * HID), lambda i: (i // NS, 0, 0)),
            pl.BlockSpec((1, 6, 16 * HID, 4 * HID),
                         lambda i: (i // NS, 0, 0, 0)),
            pl.BlockSpec((1, 6, 1, 4 * HID), lambda i: (i // NS, 0, 0, 0)),
            pl.BlockSpec((1, 6, 1, 4 * HID), lambda i: (i // NS, 0, 0, 0)),
        ],
        out_specs=pl.BlockSpec((1, BG, HID), lambda i: (i, 0, 0)),
        scratch_shapes=[
            pltpu.VMEM((BG, C0, HW + 2, HW), jnp.float32),
            pltpu.VMEM((BG * 128, 2 * _KH), jnp.bfloat16),
            pltpu.VMEM((BG, 2, 34, _OFF[6], 2 * HID), jnp.float32),
            pltpu.VMEM((BG, 32, 32, HID), jnp.float32),
            pltpu.VMEM((BG * 1024, 16 * HID), jnp.bfloat16),
        ],
        compiler_params=pltpu.CompilerParams(
            dimension_semantics=("parallel",),
            vmem_limit_bytes=64 * 1024 * 1024),
    )(x, band, band, s1t, b1t, wt4, st4, bt4)

    f = feats.reshape(L, B, HID).transpose(1, 0, 2).reshape(B, L * HID)
    out = jnp.dot(f, wc, precision=jax.lax.Precision.HIGHEST) + bc
    return out.reshape(B, 1, 1, 1)
